# baseline (speedup 1.0000x reference)
import functools

import jax
import jax.numpy as jnp
from jax import lax
from jax.experimental import pallas as pl
from jax.experimental.pallas import tpu as pltpu
from jax.experimental.pallas import tpu_sc as plsc

D = 128
NC = 2
NS = 16
NW = NC * NS


def _gather_body(idx_hbm, out_x, out_y, table_hbm, idx_v, rows_v, sem, wid,
                 n_per_tile, chunk, n_chunks):
    base = wid * n_per_tile

    def one_chunk(off):
        pltpu.sync_copy(idx_hbm.at[pl.ds(off, chunk)], idx_v)
        pltpu.async_copy(table_hbm.at[idx_v], rows_v, sem).wait()
        cx = pltpu.async_copy(rows_v, out_x.at[pl.ds(off, chunk)], sem)
        cy = pltpu.async_copy(rows_v, out_y.at[pl.ds(off, chunk)], sem)
        cx.wait()
        cy.wait()

    if n_chunks <= 2:
        for k in range(n_chunks):
            one_chunk(base + k * chunk)
    else:
        def body(j, carry):
            one_chunk(base + j * chunk)
            return carry
        lax.fori_loop(0, n_chunks, body, 0)


def _make_sc_gather(n_demo, n_big):
    demo_per = n_demo // NW
    big_per = n_big // NW
    mesh = plsc.VectorSubcoreMesh(core_axis_name="c", subcore_axis_name="s")

    @functools.partial(
        pl.kernel,
        mesh=mesh,
        out_type=[
            jax.ShapeDtypeStruct((n_demo, D), jnp.float32),
            jax.ShapeDtypeStruct((n_big, D), jnp.float32),
            jax.ShapeDtypeStruct((n_big, D), jnp.float32),
            jax.ShapeDtypeStruct((n_big, D), jnp.float32),
            jax.ShapeDtypeStruct((n_demo, D), jnp.float32),
            jax.ShapeDtypeStruct((n_big, D), jnp.float32),
            jax.ShapeDtypeStruct((n_big, D), jnp.float32),
            jax.ShapeDtypeStruct((n_big, D), jnp.float32),
        ],
        scratch_types=[
            pltpu.VMEM((448,), jnp.int32),
            pltpu.VMEM((448, D), jnp.float32),
            pltpu.VMEM((400,), jnp.int32),
            pltpu.VMEM((400, D), jnp.float32),
            pltpu.SemaphoreType.DMA,
        ],
    )
    def sc_gather(idx_demo, idx_med, idx_vit, idx_lab, table,
                  out_demo, out_med, out_vit, out_lab,
                  out_demo_y, out_med_y, out_vit_y, out_lab_y,
                  idx_a, rows_a, idx_b, rows_b, sem):
        wid = lax.axis_index("s") * NC + lax.axis_index("c")
        _gather_body(idx_demo, out_demo, out_demo_y, table, idx_a, rows_a,
                     sem, wid, demo_per, 448, 2)
        _gather_body(idx_med, out_med, out_med_y, table, idx_b, rows_b,
                     sem, wid, big_per, 400, 16)
        _gather_body(idx_vit, out_vit, out_vit_y, table, idx_b, rows_b,
                     sem, wid, big_per, 400, 16)
        _gather_body(idx_lab, out_lab, out_lab_y, table, idx_b, rows_b,
                     sem, wid, big_per, 400, 16)

    return sc_gather


def _proj_body(x_ref, wt_ref, b_ref, ox_ref, oy_ref):
    y = (
        jnp.dot(jnp.maximum(x_ref[...], 0.0), wt_ref[...],
                preferred_element_type=jnp.float32)
        + b_ref[...]
    )
    ox_ref[...] = y
    oy_ref[...] = y


def _project(x_flat, wt, b2, block_m):
    n = x_flat.shape[0]
    grid = (n // block_m,)
    return pl.pallas_call(
        _proj_body,
        grid=grid,
        in_specs=[
            pl.BlockSpec((block_m, D), lambda i: (i, 0)),
            pl.BlockSpec((D, D), lambda i: (0, 0)),
            pl.BlockSpec((1, D), lambda i: (0, 0)),
        ],
        out_specs=[
            pl.BlockSpec((block_m, D), lambda i: (i, 0)),
            pl.BlockSpec((block_m, D), lambda i: (i, 0)),
        ],
        out_shape=[
            jax.ShapeDtypeStruct((n, D), jnp.float32),
            jax.ShapeDtypeStruct((n, D), jnp.float32),
        ],
    )(x_flat, wt, b2)


def kernel(tensor_demo, tensor_med, tensor_vitals, tensor_labs, table, W, b):
    B, T_demo = tensor_demo.shape
    T_big = tensor_med.shape[1]
    n_demo = B * T_demo
    n_big = B * T_big

    idx_demo = tensor_demo.reshape(-1).astype(jnp.int32)
    idx_med = tensor_med.reshape(-1).astype(jnp.int32)
    idx_vit = tensor_vitals.reshape(-1).astype(jnp.int32)
    idx_lab = tensor_labs.reshape(-1).astype(jnp.int32)

    sc_gather = _make_sc_gather(n_demo, n_big)
    (emb_demo_f, emb_med_f, emb_vit_f, emb_lab_f,
     emb_demo_g, emb_med_g, emb_vit_g, emb_lab_g) = sc_gather(
        idx_demo, idx_med, idx_vit, idx_lab, table)

    wt = W.T
    b2 = b.reshape(1, D)
    proj_demo_f, proj_demo_g = _project(emb_demo_f, wt, b2, 2048)
    proj_med_f, proj_med_g = _project(emb_med_f, wt, b2, 2048)
    proj_vit_f, proj_vit_g = _project(emb_vit_f, wt, b2, 2048)
    proj_lab_f, proj_lab_g = _project(emb_lab_f, wt, b2, 2048)

    def shp(x, t):
        return x.reshape(B, t, D)

    embedding_x = (shp(emb_demo_f, T_demo), shp(emb_med_f, T_big),
                   shp(emb_vit_f, T_big), shp(emb_lab_f, T_big))
    embedding_y = (shp(emb_demo_g, T_demo), shp(emb_med_g, T_big),
                   shp(emb_vit_g, T_big), shp(emb_lab_g, T_big))
    projection_x = (shp(proj_demo_f, T_demo), shp(proj_med_f, T_big),
                    shp(proj_vit_f, T_big), shp(proj_lab_f, T_big))
    projection_y = (shp(proj_demo_g, T_demo), shp(proj_med_g, T_big),
                    shp(proj_vit_g, T_big), shp(proj_lab_g, T_big))
    return (embedding_x, projection_x, embedding_y, projection_y)


# speedup vs baseline: 1.4893x; 1.4893x over previous
import functools

import jax
import jax.numpy as jnp
from jax import lax
from jax.experimental import pallas as pl
from jax.experimental.pallas import tpu as pltpu
from jax.experimental.pallas import tpu_sc as plsc

D = 128
NC = 2
NS = 16
NW = NC * NS


def _gather_body(idx_hbm, out_hbm, table_hbm, idx_v, rows_v, sem, wid,
                 n_per_tile, chunk, n_chunks):
    base = wid * n_per_tile

    def one_chunk(off):
        pltpu.sync_copy(idx_hbm.at[pl.ds(off, chunk)], idx_v)
        pltpu.async_copy(table_hbm.at[idx_v], rows_v, sem).wait()
        pltpu.sync_copy(rows_v, out_hbm.at[pl.ds(off, chunk)])

    if n_chunks <= 2:
        for k in range(n_chunks):
            one_chunk(base + k * chunk)
    else:
        def body(j, carry):
            one_chunk(base + j * chunk)
            return carry
        lax.fori_loop(0, n_chunks, body, 0)


def _make_sc_gather(n_demo, n_big):
    demo_per = n_demo // NW
    big_per = n_big // NW
    mesh = plsc.VectorSubcoreMesh(core_axis_name="c", subcore_axis_name="s")

    @functools.partial(
        pl.kernel,
        mesh=mesh,
        out_type=[
            jax.ShapeDtypeStruct((n_demo, D), jnp.float32),
            jax.ShapeDtypeStruct((n_big, D), jnp.float32),
            jax.ShapeDtypeStruct((n_big, D), jnp.float32),
            jax.ShapeDtypeStruct((n_big, D), jnp.float32),
        ],
        scratch_types=[
            pltpu.VMEM((448,), jnp.int32),
            pltpu.VMEM((448, D), jnp.float32),
            pltpu.VMEM((400,), jnp.int32),
            pltpu.VMEM((400, D), jnp.float32),
            pltpu.SemaphoreType.DMA,
        ],
    )
    def sc_gather(idx_demo, idx_med, idx_vit, idx_lab, table,
                  out_demo, out_med, out_vit, out_lab,
                  idx_a, rows_a, idx_b, rows_b, sem):
        wid = lax.axis_index("s") * NC + lax.axis_index("c")
        _gather_body(idx_demo, out_demo, table, idx_a, rows_a, sem, wid,
                     demo_per, 448, 2)
        _gather_body(idx_med, out_med, table, idx_b, rows_b, sem, wid,
                     big_per, 400, 16)
        _gather_body(idx_vit, out_vit, table, idx_b, rows_b, sem, wid,
                     big_per, 400, 16)
        _gather_body(idx_lab, out_lab, table, idx_b, rows_b, sem, wid,
                     big_per, 400, 16)

    return sc_gather


def _make_proj_body(bb, t):
    def _proj_body(x_ref, wt_ref, b_ref, ex_ref, ey_ref, px_ref, py_ref):
        x = x_ref[...]
        y = (
            jnp.dot(jnp.maximum(x, 0.0), wt_ref[...],
                    preferred_element_type=jnp.float32)
            + b_ref[...]
        )
        e3 = x.reshape(bb, t, D)
        y3 = y.reshape(bb, t, D)
        ex_ref[...] = e3
        ey_ref[...] = e3
        px_ref[...] = y3
        py_ref[...] = y3
    return _proj_body


def _expand_project(x_flat, wt, b2, batch, t, bb):
    grid = (batch // bb,)
    out3 = jax.ShapeDtypeStruct((batch, t, D), jnp.float32)
    spec3 = pl.BlockSpec((bb, t, D), lambda i: (i, 0, 0))
    return pl.pallas_call(
        _make_proj_body(bb, t),
        grid=grid,
        in_specs=[
            pl.BlockSpec((bb * t, D), lambda i: (i, 0)),
            pl.BlockSpec((D, D), lambda i: (0, 0)),
            pl.BlockSpec((1, D), lambda i: (0, 0)),
        ],
        out_specs=[spec3, spec3, spec3, spec3],
        out_shape=[out3, out3, out3, out3],
    )(x_flat, wt, b2)


def kernel(tensor_demo, tensor_med, tensor_vitals, tensor_labs, table, W, b):
    B, T_demo = tensor_demo.shape
    T_big = tensor_med.shape[1]
    n_demo = B * T_demo
    n_big = B * T_big

    idx_demo = tensor_demo.reshape(-1).astype(jnp.int32)
    idx_med = tensor_med.reshape(-1).astype(jnp.int32)
    idx_vit = tensor_vitals.reshape(-1).astype(jnp.int32)
    idx_lab = tensor_labs.reshape(-1).astype(jnp.int32)

    sc_gather = _make_sc_gather(n_demo, n_big)
    emb_demo_f, emb_med_f, emb_vit_f, emb_lab_f = sc_gather(
        idx_demo, idx_med, idx_vit, idx_lab, table)

    wt = W.T
    b2 = b.reshape(1, D)
    ed_x, ed_y, pd_x, pd_y = _expand_project(emb_demo_f, wt, b2, B, T_demo, 32)
    em_x, em_y, pm_x, pm_y = _expand_project(emb_med_f, wt, b2, B, T_big, 32)
    ev_x, ev_y, pv_x, pv_y = _expand_project(emb_vit_f, wt, b2, B, T_big, 32)
    el_x, el_y, pl_x, pl_y = _expand_project(emb_lab_f, wt, b2, B, T_big, 32)

    embedding_x = (ed_x, em_x, ev_x, el_x)
    embedding_y = (ed_y, em_y, ev_y, el_y)
    projection_x = (pd_x, pm_x, pv_x, pl_x)
    projection_y = (pd_y, pm_y, pv_y, pl_y)
    return (embedding_x, projection_x, embedding_y, projection_y)


# speedup vs baseline: 1.5290x; 1.0267x over previous
import functools

import jax
import jax.numpy as jnp
from jax import lax
from jax.experimental import pallas as pl
from jax.experimental.pallas import tpu as pltpu
from jax.experimental.pallas import tpu_sc as plsc

D = 128
NC = 2
NS = 16
NW = NC * NS
MESH = dict(core_axis_name="c", subcore_axis_name="s")


def _wid():
    return lax.axis_index("s") * NC + lax.axis_index("c")


def _make_sc_gather_flat(n_rows, chunk, n_chunks):
    @functools.partial(
        pl.kernel,
        mesh=plsc.VectorSubcoreMesh(**MESH),
        out_type=jax.ShapeDtypeStruct((n_rows, D), jnp.float32),
        scratch_types=[
            pltpu.VMEM((chunk,), jnp.int32),
            pltpu.VMEM((chunk, D), jnp.float32),
            pltpu.SemaphoreType.DMA,
        ],
    )
    def sc_gather(idx_hbm, table, out_hbm, idx_v, rows_v, sem):
        base = _wid() * (n_rows // NW)

        def one_chunk(off):
            pltpu.sync_copy(idx_hbm.at[pl.ds(off, chunk)], idx_v)
            pltpu.async_copy(table.at[idx_v], rows_v, sem).wait()
            pltpu.sync_copy(rows_v, out_hbm.at[pl.ds(off, chunk)])

        if n_chunks <= 2:
            for k in range(n_chunks):
                one_chunk(base + k * chunk)
        else:
            lax.fori_loop(
                0, n_chunks,
                lambda j, c: (one_chunk(base + j * chunk), c)[1], 0)

    return sc_gather


def _make_sc_gather3d(batch, t, nb, n_chunks):
    @functools.partial(
        pl.kernel,
        mesh=plsc.VectorSubcoreMesh(**MESH),
        out_type=[
            jax.ShapeDtypeStruct((batch, t, D), jnp.float32),
            jax.ShapeDtypeStruct((batch, t, D), jnp.float32),
        ],
        scratch_types=[
            pltpu.VMEM((nb * t,), jnp.int32),
            pltpu.VMEM((nb * t, D), jnp.float32),
            pltpu.SemaphoreType.DMA,
            pltpu.SemaphoreType.DMA,
        ],
    )
    def sc_gather(idx_hbm, table, out_x, out_y, idx_v, rows_v, sem, semw):
        tile_base = _wid() * (batch // NW)

        def one_chunk(b0):
            pltpu.sync_copy(idx_hbm.at[pl.ds(b0 * t, nb * t)], idx_v)
            pltpu.async_copy(table.at[idx_v], rows_v, sem).wait()
            copies = []
            for k in range(nb):
                src = rows_v.at[pl.ds(k * t, t)]
                copies.append(pltpu.async_copy(src, out_x.at[b0 + k], semw))
                copies.append(pltpu.async_copy(src, out_y.at[b0 + k], semw))
            for c in copies:
                c.wait()

        lax.fori_loop(
            0, n_chunks,
            lambda j, c: (one_chunk(tile_base + j * nb), c)[1], 0)

    return sc_gather


def _make_expand_proj_body(bb, t):
    def body(x_ref, wt_ref, b_ref, ex_ref, ey_ref, px_ref, py_ref):
        x = x_ref[...]
        y = (
            jnp.dot(jnp.maximum(x, 0.0), wt_ref[...],
                    preferred_element_type=jnp.float32)
            + b_ref[...]
        )
        e3 = x.reshape(bb, t, D)
        y3 = y.reshape(bb, t, D)
        ex_ref[...] = e3
        ey_ref[...] = e3
        px_ref[...] = y3
        py_ref[...] = y3
    return body


def _expand_project(x_flat, wt, b2, batch, t, bb):
    out3 = jax.ShapeDtypeStruct((batch, t, D), jnp.float32)
    spec3 = pl.BlockSpec((bb, t, D), lambda i: (i, 0, 0))
    return pl.pallas_call(
        _make_expand_proj_body(bb, t),
        grid=(batch // bb,),
        in_specs=[
            pl.BlockSpec((bb * t, D), lambda i: (i, 0)),
            pl.BlockSpec((D, D), lambda i: (0, 0)),
            pl.BlockSpec((1, D), lambda i: (0, 0)),
        ],
        out_specs=[spec3, spec3, spec3, spec3],
        out_shape=[out3, out3, out3, out3],
    )(x_flat, wt, b2)


def _make_proj3d_body(bb, t):
    def body(x_ref, wt_ref, b_ref, px_ref, py_ref):
        x = x_ref[...].reshape(bb * t, D)
        y = (
            jnp.dot(jnp.maximum(x, 0.0), wt_ref[...],
                    preferred_element_type=jnp.float32)
            + b_ref[...]
        )
        y3 = y.reshape(bb, t, D)
        px_ref[...] = y3
        py_ref[...] = y3
    return body


def _project3d(emb3, wt, b2, batch, t, bb):
    out3 = jax.ShapeDtypeStruct((batch, t, D), jnp.float32)
    spec3 = pl.BlockSpec((bb, t, D), lambda i: (i, 0, 0))
    return pl.pallas_call(
        _make_proj3d_body(bb, t),
        grid=(batch // bb,),
        in_specs=[
            spec3,
            pl.BlockSpec((D, D), lambda i: (0, 0)),
            pl.BlockSpec((1, D), lambda i: (0, 0)),
        ],
        out_specs=[spec3, spec3],
        out_shape=[out3, out3],
    )(emb3, wt, b2)


def kernel(tensor_demo, tensor_med, tensor_vitals, tensor_labs, table, W, b):
    B, T_demo = tensor_demo.shape
    T_big = tensor_med.shape[1]
    n_demo = B * T_demo

    idx_demo = tensor_demo.reshape(-1).astype(jnp.int32)
    idx_med = tensor_med.reshape(-1).astype(jnp.int32)
    idx_vit = tensor_vitals.reshape(-1).astype(jnp.int32)
    idx_lab = tensor_labs.reshape(-1).astype(jnp.int32)

    gather_demo = _make_sc_gather_flat(n_demo, 448, 2)
    gather_big = _make_sc_gather3d(B, T_big, 8, (B // NW) // 8)

    emb_demo_f = gather_demo(idx_demo, table)
    em_x, em_y = gather_big(idx_med, table)
    ev_x, ev_y = gather_big(idx_vit, table)
    el_x, el_y = gather_big(idx_lab, table)

    wt = W.T
    b2 = b.reshape(1, D)
    ed_x, ed_y, pd_x, pd_y = _expand_project(emb_demo_f, wt, b2, B, T_demo, 32)
    pm_x, pm_y = _project3d(em_x, wt, b2, B, T_big, 32)
    pv_x, pv_y = _project3d(ev_x, wt, b2, B, T_big, 32)
    pl_x, pl_y = _project3d(el_x, wt, b2, B, T_big, 32)

    embedding_x = (ed_x, em_x, ev_x, el_x)
    embedding_y = (ed_y, em_y, ev_y, el_y)
    projection_x = (pd_x, pm_x, pv_x, pl_x)
    projection_y = (pd_y, pm_y, pv_y, pl_y)
    return (embedding_x, projection_x, embedding_y, projection_y)


# speedup vs baseline: 1.6525x; 1.0808x over previous
import functools

import jax
import jax.numpy as jnp
from jax import lax
from jax.experimental import pallas as pl
from jax.experimental.pallas import tpu as pltpu
from jax.experimental.pallas import tpu_sc as plsc

D = 128
NC = 2
NS = 16
NW = NC * NS


def _make_sc_gather_flat(n_rows, chunk, n_chunks):
    @functools.partial(
        pl.kernel,
        mesh=plsc.VectorSubcoreMesh(core_axis_name="c", subcore_axis_name="s"),
        out_type=jax.ShapeDtypeStruct((n_rows, D), jnp.float32),
        scratch_types=[
            pltpu.VMEM((chunk,), jnp.int32),
            pltpu.VMEM((chunk, D), jnp.float32),
            pltpu.SemaphoreType.DMA,
        ],
    )
    def sc_gather(idx_hbm, table, out_hbm, idx_v, rows_v, sem):
        wid = lax.axis_index("s") * NC + lax.axis_index("c")
        base = wid * (n_rows // NW)

        def one_chunk(off):
            pltpu.sync_copy(idx_hbm.at[pl.ds(off, chunk)], idx_v)
            pltpu.async_copy(table.at[idx_v], rows_v, sem).wait()
            pltpu.sync_copy(rows_v, out_hbm.at[pl.ds(off, chunk)])

        if n_chunks <= 2:
            for k in range(n_chunks):
                one_chunk(base + k * chunk)
        else:
            lax.fori_loop(
                0, n_chunks,
                lambda j, c: (one_chunk(base + j * chunk), c)[1], 0)

    return sc_gather


def _make_expand_proj_body(bb, t):
    def body(x_ref, wt_ref, b_ref, ex_ref, ey_ref, px_ref, py_ref):
        x = x_ref[...]
        y = (
            jnp.dot(jnp.maximum(x, 0.0), wt_ref[...],
                    preferred_element_type=jnp.float32)
            + b_ref[...]
        )
        e3 = x.reshape(bb, t, D)
        y3 = y.reshape(bb, t, D)
        ex_ref[...] = e3
        ey_ref[...] = e3
        px_ref[...] = y3
        py_ref[...] = y3
    return body


def _expand_project(x_flat, wt, b2, batch, t, bb):
    out3 = jax.ShapeDtypeStruct((batch, t, D), jnp.float32)
    spec3 = pl.BlockSpec((bb, t, D), lambda i: (i, 0, 0))
    return pl.pallas_call(
        _make_expand_proj_body(bb, t),
        grid=(batch // bb,),
        in_specs=[
            pl.BlockSpec((bb * t, D), lambda i: (i, 0)),
            pl.BlockSpec((D, D), lambda i: (0, 0)),
            pl.BlockSpec((1, D), lambda i: (0, 0)),
        ],
        out_specs=[spec3, spec3, spec3, spec3],
        out_shape=[out3, out3, out3, out3],
    )(x_flat, wt, b2)


def kernel(tensor_demo, tensor_med, tensor_vitals, tensor_labs, table, W, b):
    B, T_demo = tensor_demo.shape
    T_big = tensor_med.shape[1]
    n_demo = B * T_demo
    n_big = B * T_big

    idx_demo = tensor_demo.reshape(-1).astype(jnp.int32)
    idx_med = tensor_med.reshape(-1).astype(jnp.int32)
    idx_vit = tensor_vitals.reshape(-1).astype(jnp.int32)
    idx_lab = tensor_labs.reshape(-1).astype(jnp.int32)

    gather_demo = _make_sc_gather_flat(n_demo, 448, 2)
    gather_big = _make_sc_gather_flat(n_big, 400, 16)

    emb_demo_f = gather_demo(idx_demo, table)
    emb_med_f = gather_big(idx_med, table)
    emb_vit_f = gather_big(idx_vit, table)
    emb_lab_f = gather_big(idx_lab, table)

    wt = W.T
    b2 = b.reshape(1, D)
    ed_x, ed_y, pd_x, pd_y = _expand_project(emb_demo_f, wt, b2, B, T_demo, 64)
    em_x, em_y, pm_x, pm_y = _expand_project(emb_med_f, wt, b2, B, T_big, 64)
    ev_x, ev_y, pv_x, pv_y = _expand_project(emb_vit_f, wt, b2, B, T_big, 64)
    el_x, el_y, pl_x, pl_y = _expand_project(emb_lab_f, wt, b2, B, T_big, 64)

    embedding_x = (ed_x, em_x, ev_x, el_x)
    embedding_y = (ed_y, em_y, ev_y, el_y)
    projection_x = (pd_x, pm_x, pv_x, pl_x)
    projection_y = (pd_y, pm_y, pv_y, pl_y)
    return (embedding_x, projection_x, embedding_y, projection_y)


# speedup vs baseline: 3.8051x; 2.3026x over previous
import functools

import jax
import jax.numpy as jnp
from jax import lax
from jax.experimental import pallas as pl
from jax.experimental.pallas import tpu as pltpu
from jax.experimental.pallas import tpu_sc as plsc

D = 128
NC = 2
NS = 16
NW = NC * NS


def _make_sc_gather_dual(n_rows, chunk, n_chunks):
    @functools.partial(
        pl.kernel,
        mesh=plsc.VectorSubcoreMesh(core_axis_name="c", subcore_axis_name="s"),
        out_type=[
            jax.ShapeDtypeStruct((n_rows, D), jnp.float32),
            jax.ShapeDtypeStruct((n_rows, D), jnp.float32),
        ],
        scratch_types=[
            pltpu.VMEM((chunk,), jnp.int32),
            pltpu.VMEM((chunk, D), jnp.float32),
            pltpu.SemaphoreType.DMA,
        ],
    )
    def sc_gather(idx_hbm, table, out_x, out_y, idx_v, rows_v, sem):
        wid = lax.axis_index("s") * NC + lax.axis_index("c")
        base = wid * (n_rows // NW)

        def one_chunk(off):
            pltpu.sync_copy(idx_hbm.at[pl.ds(off, chunk)], idx_v)
            pltpu.async_copy(table.at[idx_v], rows_v, sem).wait()
            cx = pltpu.async_copy(rows_v, out_x.at[pl.ds(off, chunk)], sem)
            cy = pltpu.async_copy(rows_v, out_y.at[pl.ds(off, chunk)], sem)
            cx.wait()
            cy.wait()

        if n_chunks <= 2:
            for k in range(n_chunks):
                one_chunk(base + k * chunk)
        else:
            lax.fori_loop(
                0, n_chunks,
                lambda j, c: (one_chunk(base + j * chunk), c)[1], 0)

    return sc_gather


def _proj_body(x_ref, wt_ref, b_ref, px_ref, py_ref):
    y = (
        jnp.dot(jnp.maximum(x_ref[...], 0.0), wt_ref[...],
                preferred_element_type=jnp.float32)
        + b_ref[...]
    )
    px_ref[...] = y
    py_ref[...] = y


def _project(x_flat, wt, b2, bm):
    n = x_flat.shape[0]
    out2 = jax.ShapeDtypeStruct((n, D), jnp.float32)
    spec = pl.BlockSpec((bm, D), lambda i: (i, 0))
    return pl.pallas_call(
        _proj_body,
        grid=(n // bm,),
        in_specs=[
            spec,
            pl.BlockSpec((D, D), lambda i: (0, 0)),
            pl.BlockSpec((1, D), lambda i: (0, 0)),
        ],
        out_specs=[spec, spec],
        out_shape=[out2, out2],
    )(x_flat, wt, b2)


def kernel(tensor_demo, tensor_med, tensor_vitals, tensor_labs, table, W, b):
    B, T_demo = tensor_demo.shape
    T_big = tensor_med.shape[1]
    n_demo = B * T_demo
    n_big = B * T_big

    def tmaj(x):
        return x.T.reshape(-1).astype(jnp.int32)

    idx_demo = tmaj(tensor_demo)
    idx_med = tmaj(tensor_med)
    idx_vit = tmaj(tensor_vitals)
    idx_lab = tmaj(tensor_labs)

    gather_demo = _make_sc_gather_dual(n_demo, 448, 2)
    gather_big = _make_sc_gather_dual(n_big, 400, 16)

    ed_fx, ed_fy = gather_demo(idx_demo, table)
    em_fx, em_fy = gather_big(idx_med, table)
    ev_fx, ev_fy = gather_big(idx_vit, table)
    el_fx, el_fy = gather_big(idx_lab, table)

    wt = W.T
    b2 = b.reshape(1, D)
    pd_fx, pd_fy = _project(ed_fx, wt, b2, 2048)
    pm_fx, pm_fy = _project(em_fx, wt, b2, 2048)
    pv_fx, pv_fy = _project(ev_fx, wt, b2, 2048)
    pl_fx, pl_fy = _project(el_fx, wt, b2, 2048)

    def btd(x_flat, t):
        return jnp.transpose(x_flat.reshape(t, B, D), (1, 0, 2))

    embedding_x = (btd(ed_fx, T_demo), btd(em_fx, T_big),
                   btd(ev_fx, T_big), btd(el_fx, T_big))
    embedding_y = (btd(ed_fy, T_demo), btd(em_fy, T_big),
                   btd(ev_fy, T_big), btd(el_fy, T_big))
    projection_x = (btd(pd_fx, T_demo), btd(pm_fx, T_big),
                    btd(pv_fx, T_big), btd(pl_fx, T_big))
    projection_y = (btd(pd_fy, T_demo), btd(pm_fy, T_big),
                    btd(pv_fy, T_big), btd(pl_fy, T_big))
    return (embedding_x, projection_x, embedding_y, projection_y)


# speedup vs baseline: 3.9359x; 1.0344x over previous
import functools

import jax
import jax.numpy as jnp
from jax import lax
from jax.experimental import pallas as pl
from jax.experimental.pallas import tpu as pltpu
from jax.experimental.pallas import tpu_sc as plsc

D = 128
NC = 2
NS = 16
NW = NC * NS


def _make_sc_gather_dual(n_rows, chunk, n_chunks):
    @functools.partial(
        pl.kernel,
        mesh=plsc.VectorSubcoreMesh(core_axis_name="c", subcore_axis_name="s"),
        out_type=[
            jax.ShapeDtypeStruct((n_rows, D), jnp.float32),
            jax.ShapeDtypeStruct((n_rows, D), jnp.float32),
        ],
        scratch_types=[
            pltpu.VMEM((chunk,), jnp.int32),
            pltpu.VMEM((chunk, D), jnp.float32),
            pltpu.SemaphoreType.DMA,
        ],
    )
    def sc_gather(idx_hbm, table, out_x, out_y, idx_v, rows_v, sem):
        wid = lax.axis_index("s") * NC + lax.axis_index("c")
        base = wid * (n_rows // NW)

        def one_chunk(off):
            pltpu.sync_copy(idx_hbm.at[pl.ds(off, chunk)], idx_v)
            pltpu.async_copy(table.at[idx_v], rows_v, sem).wait()
            cx = pltpu.async_copy(rows_v, out_x.at[pl.ds(off, chunk)], sem)
            cy = pltpu.async_copy(rows_v, out_y.at[pl.ds(off, chunk)], sem)
            cx.wait()
            cy.wait()

        if n_chunks <= 2:
            for k in range(n_chunks):
                one_chunk(base + k * chunk)
        else:
            lax.fori_loop(
                0, n_chunks,
                lambda j, c: (one_chunk(base + j * chunk), c)[1], 0)

    return sc_gather


def _proj_body(x_ref, wt_ref, b_ref, px_ref, py_ref):
    y = (
        jnp.dot(jnp.maximum(x_ref[...], 0.0), wt_ref[...],
                preferred_element_type=jnp.float32)
        + b_ref[...]
    )
    px_ref[...] = y
    py_ref[...] = y


def _project(x_flat, wt, b2, bm):
    n = x_flat.shape[0]
    out2 = jax.ShapeDtypeStruct((n, D), jnp.float32)
    spec = pl.BlockSpec((bm, D), lambda i: (i, 0))
    return pl.pallas_call(
        _proj_body,
        grid=(n // bm,),
        in_specs=[
            spec,
            pl.BlockSpec((D, D), lambda i: (0, 0)),
            pl.BlockSpec((1, D), lambda i: (0, 0)),
        ],
        out_specs=[spec, spec],
        out_shape=[out2, out2],
    )(x_flat, wt, b2)


def kernel(tensor_demo, tensor_med, tensor_vitals, tensor_labs, table, W, b):
    B, T_demo = tensor_demo.shape
    T_big = tensor_med.shape[1]
    n_demo = B * T_demo
    n_big = B * T_big

    def tmaj(x):
        return x.T.reshape(-1).astype(jnp.int32)

    idx_demo = tmaj(tensor_demo)
    idx_med = tmaj(tensor_med)
    idx_vit = tmaj(tensor_vitals)
    idx_lab = tmaj(tensor_labs)

    gather_demo = _make_sc_gather_dual(n_demo, 448, 2)
    gather_big = _make_sc_gather_dual(n_big, 400, 16)

    em_fx, em_fy = gather_big(idx_med, table)
    ev_fx, ev_fy = gather_big(idx_vit, table)
    el_fx, el_fy = gather_big(idx_lab, table)
    ed_fx, ed_fy = gather_demo(idx_demo, table)

    wt = W.T
    b2 = b.reshape(1, D)
    pm_fx, pm_fy = _project(em_fx, wt, b2, 4096)
    pv_fx, pv_fy = _project(ev_fx, wt, b2, 4096)
    pl_fx, pl_fy = _project(el_fx, wt, b2, 4096)
    pd_fx, pd_fy = _project(ed_fx, wt, b2, 4096)

    def btd(x_flat, t):
        return jnp.transpose(x_flat.reshape(t, B, D), (1, 0, 2))

    embedding_x = (btd(ed_fx, T_demo), btd(em_fx, T_big),
                   btd(ev_fx, T_big), btd(el_fx, T_big))
    embedding_y = (btd(ed_fy, T_demo), btd(em_fy, T_big),
                   btd(ev_fy, T_big), btd(el_fy, T_big))
    projection_x = (btd(pd_fx, T_demo), btd(pm_fx, T_big),
                    btd(pv_fx, T_big), btd(pl_fx, T_big))
    projection_y = (btd(pd_fy, T_demo), btd(pm_fy, T_big),
                    btd(pv_fy, T_big), btd(pl_fy, T_big))
    return (embedding_x, projection_x, embedding_y, projection_y)


# speedup vs baseline: 4.0059x; 1.0178x over previous
import functools

import jax
import jax.numpy as jnp
from jax import lax
from jax.experimental import pallas as pl
from jax.experimental.pallas import tpu as pltpu
from jax.experimental.pallas import tpu_sc as plsc

D = 128
NC = 2
NS = 16
NW = NC * NS


def _make_sc_gather_dual(n_rows, chunk, n_chunks):
    @functools.partial(
        pl.kernel,
        mesh=plsc.VectorSubcoreMesh(core_axis_name="c", subcore_axis_name="s"),
        out_type=[
            jax.ShapeDtypeStruct((n_rows, D), jnp.float32),
            jax.ShapeDtypeStruct((n_rows, D), jnp.float32),
        ],
        scratch_types=[
            pltpu.VMEM((chunk,), jnp.int32),
            pltpu.VMEM((chunk, D), jnp.float32),
            pltpu.SemaphoreType.DMA,
        ],
    )
    def sc_gather(idx_hbm, table, out_x, out_y, idx_v, rows_v, sem):
        wid = lax.axis_index("s") * NC + lax.axis_index("c")
        base = wid * (n_rows // NW)

        def one_chunk(off):
            pltpu.sync_copy(idx_hbm.at[pl.ds(off, chunk)], idx_v)
            pltpu.async_copy(table.at[idx_v], rows_v, sem).wait()
            cx = pltpu.async_copy(rows_v, out_x.at[pl.ds(off, chunk)], sem)
            cy = pltpu.async_copy(rows_v, out_y.at[pl.ds(off, chunk)], sem)
            cx.wait()
            cy.wait()

        if n_chunks <= 2:
            for k in range(n_chunks):
                one_chunk(base + k * chunk)
        else:
            lax.fori_loop(
                0, n_chunks,
                lambda j, c: (one_chunk(base + j * chunk), c)[1], 0)

    return sc_gather


def _proj_body(x_ref, wt_ref, b_ref, px_ref, py_ref):
    y = (
        jnp.dot(jnp.maximum(x_ref[...], 0.0), wt_ref[...],
                preferred_element_type=jnp.float32)
        + b_ref[...]
    )
    px_ref[...] = y
    py_ref[...] = y


def _project(x_flat, wt, b2, bm):
    n = x_flat.shape[0]
    out2 = jax.ShapeDtypeStruct((n, D), jnp.float32)
    spec = pl.BlockSpec((bm, D), lambda i: (i, 0))
    return pl.pallas_call(
        _proj_body,
        grid=(n // bm,),
        in_specs=[
            spec,
            pl.BlockSpec((D, D), lambda i: (0, 0)),
            pl.BlockSpec((1, D), lambda i: (0, 0)),
        ],
        out_specs=[spec, spec],
        out_shape=[out2, out2],
    )(x_flat, wt, b2)


def kernel(tensor_demo, tensor_med, tensor_vitals, tensor_labs, table, W, b):
    B, T_demo = tensor_demo.shape
    T_big = tensor_med.shape[1]
    n_demo = B * T_demo
    n_big = B * T_big

    def tmaj(x):
        return x.T.reshape(-1).astype(jnp.int32)

    idx_demo = tmaj(tensor_demo)
    idx_med = tmaj(tensor_med)
    idx_vit = tmaj(tensor_vitals)
    idx_lab = tmaj(tensor_labs)

    gather_demo = _make_sc_gather_dual(n_demo, 448, 2)
    gather_big = _make_sc_gather_dual(n_big, 800, 8)

    em_fx, em_fy = gather_big(idx_med, table)
    ev_fx, ev_fy = gather_big(idx_vit, table)
    el_fx, el_fy = gather_big(idx_lab, table)
    ed_fx, ed_fy = gather_demo(idx_demo, table)

    wt = W.T
    b2 = b.reshape(1, D)
    pm_fx, pm_fy = _project(em_fx, wt, b2, 4096)
    pv_fx, pv_fy = _project(ev_fx, wt, b2, 4096)
    pl_fx, pl_fy = _project(el_fx, wt, b2, 4096)
    pd_fx, pd_fy = _project(ed_fx, wt, b2, 4096)

    def btd(x_flat, t):
        return jnp.transpose(x_flat.reshape(t, B, D), (1, 0, 2))

    embedding_x = (btd(ed_fx, T_demo), btd(em_fx, T_big),
                   btd(ev_fx, T_big), btd(el_fx, T_big))
    embedding_y = (btd(ed_fy, T_demo), btd(em_fy, T_big),
                   btd(ev_fy, T_big), btd(el_fy, T_big))
    projection_x = (btd(pd_fx, T_demo), btd(pm_fx, T_big),
                    btd(pv_fx, T_big), btd(pl_fx, T_big))
    projection_y = (btd(pd_fy, T_demo), btd(pm_fy, T_big),
                    btd(pv_fy, T_big), btd(pl_fy, T_big))
    return (embedding_x, projection_x, embedding_y, projection_y)


# speedup vs baseline: 4.0084x; 1.0006x over previous
import functools

import jax
import jax.numpy as jnp
from jax import lax
from jax.experimental import pallas as pl
from jax.experimental.pallas import tpu as pltpu
from jax.experimental.pallas import tpu_sc as plsc

D = 128
NC = 2
NS = 16
NW = NC * NS


def _make_sc_gather_dual(n_rows, chunk, n_chunks):
    assert n_chunks % 2 == 0
    @functools.partial(
        pl.kernel,
        mesh=plsc.VectorSubcoreMesh(core_axis_name="c", subcore_axis_name="s"),
        out_type=[
            jax.ShapeDtypeStruct((n_rows, D), jnp.float32),
            jax.ShapeDtypeStruct((n_rows, D), jnp.float32),
        ],
        scratch_types=[
            pltpu.VMEM((chunk,), jnp.int32),
            pltpu.VMEM((chunk,), jnp.int32),
            pltpu.VMEM((chunk, D), jnp.float32),
            pltpu.VMEM((chunk, D), jnp.float32),
            pltpu.SemaphoreType.DMA,
            pltpu.SemaphoreType.DMA,
        ],
    )
    def sc_gather(idx_hbm, table, out_x, out_y,
                  idx_a, idx_b, rows_a, rows_b, semg, semw):
        wid = lax.axis_index("s") * NC + lax.axis_index("c")
        base = wid * (n_rows // NW)

        def two_chunks(off0):
            off1 = off0 + chunk
            pltpu.sync_copy(idx_hbm.at[pl.ds(off0, chunk)], idx_a)
            ga = pltpu.async_copy(table.at[idx_a], rows_a, semg)
            pltpu.sync_copy(idx_hbm.at[pl.ds(off1, chunk)], idx_b)
            gb = pltpu.async_copy(table.at[idx_b], rows_b, semg)
            ga.wait()
            wxa = pltpu.async_copy(rows_a, out_x.at[pl.ds(off0, chunk)], semw)
            wya = pltpu.async_copy(rows_a, out_y.at[pl.ds(off0, chunk)], semw)
            gb.wait()
            wxb = pltpu.async_copy(rows_b, out_x.at[pl.ds(off1, chunk)], semw)
            wyb = pltpu.async_copy(rows_b, out_y.at[pl.ds(off1, chunk)], semw)
            wxa.wait()
            wya.wait()
            wxb.wait()
            wyb.wait()

        if n_chunks <= 2:
            two_chunks(base)
        else:
            lax.fori_loop(
                0, n_chunks // 2,
                lambda j, c: (two_chunks(base + j * 2 * chunk), c)[1], 0)

    return sc_gather


def _proj_body(x_ref, wt_ref, b_ref, px_ref, py_ref):
    y = (
        jnp.dot(jnp.maximum(x_ref[...], 0.0), wt_ref[...],
                preferred_element_type=jnp.float32)
        + b_ref[...]
    )
    px_ref[...] = y
    py_ref[...] = y


def _project(x_flat, wt, b2, bm):
    n = x_flat.shape[0]
    out2 = jax.ShapeDtypeStruct((n, D), jnp.float32)
    spec = pl.BlockSpec((bm, D), lambda i: (i, 0))
    return pl.pallas_call(
        _proj_body,
        grid=(n // bm,),
        in_specs=[
            spec,
            pl.BlockSpec((D, D), lambda i: (0, 0)),
            pl.BlockSpec((1, D), lambda i: (0, 0)),
        ],
        out_specs=[spec, spec],
        out_shape=[out2, out2],
    )(x_flat, wt, b2)


def kernel(tensor_demo, tensor_med, tensor_vitals, tensor_labs, table, W, b):
    B, T_demo = tensor_demo.shape
    T_big = tensor_med.shape[1]
    n_demo = B * T_demo
    n_big = B * T_big

    def tmaj(x):
        return x.T.reshape(-1).astype(jnp.int32)

    idx_demo = tmaj(tensor_demo)
    idx_med = tmaj(tensor_med)
    idx_vit = tmaj(tensor_vitals)
    idx_lab = tmaj(tensor_labs)

    gather_demo = _make_sc_gather_dual(n_demo, 448, 2)
    gather_big = _make_sc_gather_dual(n_big, 400, 16)

    em_fx, em_fy = gather_big(idx_med, table)
    ev_fx, ev_fy = gather_big(idx_vit, table)
    el_fx, el_fy = gather_big(idx_lab, table)
    ed_fx, ed_fy = gather_demo(idx_demo, table)

    wt = W.T
    b2 = b.reshape(1, D)
    pm_fx, pm_fy = _project(em_fx, wt, b2, 4096)
    pv_fx, pv_fy = _project(ev_fx, wt, b2, 4096)
    pl_fx, pl_fy = _project(el_fx, wt, b2, 4096)
    pd_fx, pd_fy = _project(ed_fx, wt, b2, 4096)

    def btd(x_flat, t):
        return jnp.transpose(x_flat.reshape(t, B, D), (1, 0, 2))

    embedding_x = (btd(ed_fx, T_demo), btd(em_fx, T_big),
                   btd(ev_fx, T_big), btd(el_fx, T_big))
    embedding_y = (btd(ed_fy, T_demo), btd(em_fy, T_big),
                   btd(ev_fy, T_big), btd(el_fy, T_big))
    projection_x = (btd(pd_fx, T_demo), btd(pm_fx, T_big),
                    btd(pv_fx, T_big), btd(pl_fx, T_big))
    projection_y = (btd(pd_fy, T_demo), btd(pm_fy, T_big),
                    btd(pv_fy, T_big), btd(pl_fy, T_big))
    return (embedding_x, projection_x, embedding_y, projection_y)
